# baseline (speedup 1.0000x reference)
import jax
import jax.numpy as jnp
from jax import lax
from jax.experimental import pallas as pl
from jax.experimental.pallas import tpu as pltpu
from jax.experimental.pallas import tpu_sc as plsc

E = 8
D_MODEL = 2048
D_FF = 1024
T = 2048
TOP_K = 2
NPAIR = T * TOP_K

BT = 128
BT_LOG = 7
CAP = NPAIR + E * BT
NB = CAP // BT
NB_PAD = 48

NW1 = 16
PP1 = NPAIR // NW1
NW = 32
PP = NPAIR // NW
TPT = T // NW
LANES = 16

_GDN = lax.GatherDimensionNumbers(
    offset_dims=(), collapsed_slice_dims=(0,), start_index_map=(0,))


def _vgather(vec, idx):
    return lax.gather(vec, idx[:, None], _GDN, (1,),
                      mode=lax.GatherScatterMode.PROMISE_IN_BOUNDS)


def _lt(a, b):
    return lax.shift_right_logical(a - b, 31)


def _eq(a, b):
    return 1 - jnp.minimum(jnp.abs(a - b), 1)


def _prefix(x, iota):
    for k in (1, 2, 4, 8):
        kv = iota * 0 + k
        sh = _vgather(x, jnp.maximum(iota - kv, 0))
        x = x + sh * (1 - _lt(iota, kv))
    return x


def _route_body(idx_hbm, wts_hbm, ws_hbm, be_hbm, pos_hbm, histx_hbm,
                ev_v, dsts_v, hist_v, allh_v, beh_v, dst2d, wv2d, sem):
    wid = lax.axis_index("s")
    base = wid * PP1
    iota = lax.iota(jnp.int32, LANES)
    widv = iota * 0 + wid
    c15 = iota * 0 + (LANES - 1)

    pltpu.sync_copy(idx_hbm.at[pl.ds(base, PP1)], ev_v)
    for j in range(2):
        pltpu.sync_copy(wts_hbm.at[pl.ds(base + j * PP1 // 2, PP1 // 2)],
                        wv2d.at[j])

    hist = iota * 0
    for c in range(PP1 // LANES):
        ev = ev_v[pl.ds(c * LANES, LANES)]
        for e in range(E):
            evec = iota * 0 + e
            cs = _prefix(_eq(ev, evec), iota)
            hist = hist + _vgather(cs, c15) * _eq(iota, evec)
    hist_v[...] = hist
    pltpu.sync_copy(hist_v, histx_hbm.at[wid])
    plsc.subcore_barrier()
    pltpu.sync_copy(histx_hbm, allh_v)

    count = iota * 0
    before = iota * 0
    for t in range(NW1):
        h_t = allh_v[t, :]
        count = count + h_t
        before = before + h_t * _lt(iota * 0 + t, widv)

    padded = ((count + (BT - 1)) >> BT_LOG) << BT_LOG
    startx = _prefix(padded, iota) - padded
    ends = startx + padded
    carry = startx + before

    for c in range(PP1 // LANES):
        ev = ev_v[pl.ds(c * LANES, LANES)]
        rank = iota * 0
        chist = iota * 0
        for e in range(E):
            evec = iota * 0 + e
            mi = _eq(ev, evec)
            cs = _prefix(mi, iota)
            rank = rank + mi * (cs - 1)
            chist = chist + _vgather(cs, c15) * _eq(iota, evec)
        dst = _vgather(carry, ev) + rank
        carry = carry + chist
        dsts_v[pl.ds(c * LANES, LANES)] = dst
        row, half = c // 8, (c % 8) * LANES
        dst2d[row, pl.ds(half, LANES)] = dst

    pltpu.sync_copy(dsts_v, pos_hbm.at[pl.ds(base, PP1)])
    for j in range(2):
        pltpu.async_copy(wv2d.at[j], ws_hbm.at[dst2d.at[j]], sem).wait()

    @pl.when(wid == 0)
    def _():
        for cc in range(NB_PAD // LANES):
            bid = (cc * LANES + iota) * BT
            acc = iota * 0
            for e in range(E):
                end_splat = _vgather(ends, iota * 0 + e)
                acc = acc + (1 - _lt(bid, end_splat))
            beh_v[pl.ds(cc * LANES, LANES)] = jnp.minimum(acc, E - 1)
        pltpu.sync_copy(beh_v, be_hbm)


def _gather_body(hidden_hbm, pos_hbm, xs_hbm,
                 pos_v, tok2d, dst2d, rows_v, sem):
    wid = lax.axis_index("s") * 2 + lax.axis_index("c")
    base = wid * PP
    iota = lax.iota(jnp.int32, LANES)
    widv = iota * 0 + wid

    pltpu.sync_copy(pos_hbm.at[pl.ds(base, PP)], pos_v)
    for c in range(PP // LANES):
        row, half = c // 2, (c % 2) * LANES
        dst2d[row, pl.ds(half, LANES)] = pos_v[pl.ds(c * LANES, LANES)]
        tok = (widv * PP + c * LANES + iota) & (T - 1)
        tok2d[row, pl.ds(half, LANES)] = tok

    for j in range(4):
        pltpu.async_copy(hidden_hbm.at[tok2d.at[j]], rows_v, sem).wait()
        pltpu.async_copy(rows_v, xs_hbm.at[dst2d.at[j]], sem).wait()


def _ffn_body(be_ref, ws_ref, xs_ref, gu_hbm, dp_hbm, ys_ref,
              wg_v, wd_v, sem_g, sem_d):
    i = pl.program_id(0)
    e = be_ref[i]
    prev = be_ref[jnp.maximum(i - 1, 0)]

    @pl.when((i == 0) | (e != prev))
    def _():
        cg = pltpu.make_async_copy(gu_hbm.at[e], wg_v, sem_g)
        cd = pltpu.make_async_copy(dp_hbm.at[e], wd_v, sem_d)
        cg.start()
        cd.start()
        cg.wait()
        cd.wait()

    x = xs_ref[...].astype(jnp.bfloat16)
    gate = lax.dot_general(x, wg_v[:D_FF, :],
                           (((1,), (1,)), ((), ())),
                           preferred_element_type=jnp.float32)
    up = lax.dot_general(x, wg_v[D_FF:, :],
                         (((1,), (1,)), ((), ())),
                         preferred_element_type=jnp.float32)
    h = (jax.nn.silu(gate) * up).astype(jnp.bfloat16)
    eo = lax.dot_general(h, wd_v[...],
                         (((1,), (1,)), ((), ())),
                         preferred_element_type=jnp.float32)
    w = ws_ref[0, 0, :]
    ys_ref[...] = w[:, None] * eo


def _combine_body(ys_hbm, pos_hbm, out_hbm, p0_v, p1_v, a0_v, a1_v, o_v,
                  sem, sem2):
    wid = lax.axis_index("s") * 2 + lax.axis_index("c")
    tb = wid * TPT
    pltpu.sync_copy(pos_hbm.at[pl.ds(tb, TPT)], p0_v)
    pltpu.sync_copy(pos_hbm.at[pl.ds(T + tb, TPT)], p1_v)
    for j in range(TPT // LANES):
        cp0 = pltpu.async_copy(
            ys_hbm.at[p0_v.at[pl.ds(j * LANES, LANES)]], a0_v, sem)
        cp1 = pltpu.async_copy(
            ys_hbm.at[p1_v.at[pl.ds(j * LANES, LANES)]], a1_v, sem2)
        cp0.wait()
        cp1.wait()
        for t in range(LANES):
            def _add(k, carry, t=t):
                col = k * (8 * LANES)
                for u in range(8):
                    sl = pl.ds(col + u * LANES, LANES)
                    o_v[t, sl] = a0_v[t, sl] + a1_v[t, sl]
                return carry
            lax.fori_loop(0, D_MODEL // (8 * LANES), _add, 0)
        pltpu.sync_copy(o_v, out_hbm.at[pl.ds(tb + j * LANES, LANES)])


def kernel(hidden_states, topk_indices, topk_weights, gate_up_proj, down_proj):
    idx_cm = topk_indices.astype(jnp.int32).T.reshape(NPAIR)
    wts_cm = topk_weights.T.reshape(NPAIR)
    gu_bf = gate_up_proj.astype(jnp.bfloat16)
    dp_bf = down_proj.astype(jnp.bfloat16)

    mesh1 = plsc.VectorSubcoreMesh(
        core_axis_name="c", subcore_axis_name="s", num_cores=1)
    mesh2 = plsc.VectorSubcoreMesh(
        core_axis_name="c", subcore_axis_name="s", num_cores=2)

    route = pl.kernel(
        _route_body,
        out_type=[
            jax.ShapeDtypeStruct((CAP,), jnp.float32),
            jax.ShapeDtypeStruct((NB_PAD,), jnp.int32),
            jax.ShapeDtypeStruct((NPAIR,), jnp.int32),
            jax.ShapeDtypeStruct((NW1, LANES), jnp.int32),
        ],
        mesh=mesh1,
        scratch_types=[
            pltpu.VMEM((PP1,), jnp.int32),
            pltpu.VMEM((PP1,), jnp.int32),
            pltpu.VMEM((LANES,), jnp.int32),
            pltpu.VMEM((NW1, LANES), jnp.int32),
            pltpu.VMEM((NB_PAD,), jnp.int32),
            pltpu.VMEM((2, PP1 // 2), jnp.int32),
            pltpu.VMEM((2, PP1 // 2), jnp.float32),
            pltpu.SemaphoreType.DMA,
        ],
    )
    ws, be, pos, _ = route(idx_cm, wts_cm)

    gather = pl.kernel(
        _gather_body,
        out_type=jax.ShapeDtypeStruct((CAP, D_MODEL), jnp.float32),
        mesh=mesh2,
        scratch_types=[
            pltpu.VMEM((PP,), jnp.int32),
            pltpu.VMEM((4, PP // 4), jnp.int32),
            pltpu.VMEM((4, PP // 4), jnp.int32),
            pltpu.VMEM((PP // 4, D_MODEL), jnp.float32),
            pltpu.SemaphoreType.DMA,
        ],
    )
    xs = gather(hidden_states, pos)

    grid_spec = pltpu.PrefetchScalarGridSpec(
        num_scalar_prefetch=1,
        grid=(NB,),
        in_specs=[
            pl.BlockSpec((1, 1, BT), lambda i, be_r: (i, 0, 0)),
            pl.BlockSpec((BT, D_MODEL), lambda i, be_r: (i, 0)),
            pl.BlockSpec(memory_space=pltpu.MemorySpace.HBM),
            pl.BlockSpec(memory_space=pltpu.MemorySpace.HBM),
        ],
        out_specs=pl.BlockSpec((BT, D_MODEL), lambda i, be_r: (i, 0)),
        scratch_shapes=[
            pltpu.VMEM((2 * D_FF, D_MODEL), jnp.bfloat16),
            pltpu.VMEM((D_MODEL, D_FF), jnp.bfloat16),
            pltpu.SemaphoreType.DMA,
            pltpu.SemaphoreType.DMA,
        ],
    )
    ys = pl.pallas_call(
        _ffn_body,
        grid_spec=grid_spec,
        out_shape=jax.ShapeDtypeStruct((CAP, D_MODEL), jnp.float32),
    )(be, ws.reshape(NB, 1, BT), xs, gu_bf, dp_bf)

    combine = pl.kernel(
        _combine_body,
        out_type=jax.ShapeDtypeStruct((T, D_MODEL), jnp.float32),
        mesh=mesh2,
        scratch_types=[
            pltpu.VMEM((TPT,), jnp.int32),
            pltpu.VMEM((TPT,), jnp.int32),
            pltpu.VMEM((LANES, D_MODEL), jnp.float32),
            pltpu.VMEM((LANES, D_MODEL), jnp.float32),
            pltpu.VMEM((LANES, D_MODEL), jnp.float32),
            pltpu.SemaphoreType.DMA,
            pltpu.SemaphoreType.DMA,
        ],
    )
    return combine(ys, pos)


# speedup vs baseline: 1.3513x; 1.3513x over previous
import jax
import jax.numpy as jnp
from jax import lax
from jax.experimental import pallas as pl
from jax.experimental.pallas import tpu as pltpu
from jax.experimental.pallas import tpu_sc as plsc

E = 8
D_MODEL = 2048
D_FF = 1024
T = 2048
TOP_K = 2
NPAIR = T * TOP_K

BT = 256
BT_LOG = 8
CAP = NPAIR + E * BT
NB = CAP // BT
NB_PAD = 32

NW1 = 16
PP1 = NPAIR // NW1
NW = 32
PP = NPAIR // NW
TPT = T // NW
LANES = 16

_GDN = lax.GatherDimensionNumbers(
    offset_dims=(), collapsed_slice_dims=(0,), start_index_map=(0,))


def _vgather(vec, idx):
    return lax.gather(vec, idx[:, None], _GDN, (1,),
                      mode=lax.GatherScatterMode.PROMISE_IN_BOUNDS)


def _lt(a, b):
    return lax.shift_right_logical(a - b, 31)


def _eq(a, b):
    return 1 - jnp.minimum(jnp.abs(a - b), 1)


def _prefix(x, iota):
    for k in (1, 2, 4, 8):
        kv = iota * 0 + k
        sh = _vgather(x, jnp.maximum(iota - kv, 0))
        x = x + sh * (1 - _lt(iota, kv))
    return x


def _route_body(idx_hbm, wts_hbm, ws_hbm, be_hbm, pos_hbm, histx_hbm,
                ev_v, dsts_v, hist_v, allh_v, beh_v, dst2d, wv2d, sem):
    wid = lax.axis_index("s")
    base = wid * PP1
    iota = lax.iota(jnp.int32, LANES)
    widv = iota * 0 + wid
    c15 = iota * 0 + (LANES - 1)

    pltpu.sync_copy(idx_hbm.at[pl.ds(base, PP1)], ev_v)
    for j in range(2):
        pltpu.sync_copy(wts_hbm.at[pl.ds(base + j * PP1 // 2, PP1 // 2)],
                        wv2d.at[j])

    hist = iota * 0
    for c in range(PP1 // LANES):
        ev = ev_v[pl.ds(c * LANES, LANES)]
        for e in range(E):
            evec = iota * 0 + e
            cs = _prefix(_eq(ev, evec), iota)
            hist = hist + _vgather(cs, c15) * _eq(iota, evec)
    hist_v[...] = hist
    pltpu.sync_copy(hist_v, histx_hbm.at[wid])
    plsc.subcore_barrier()
    pltpu.sync_copy(histx_hbm, allh_v)

    count = iota * 0
    before = iota * 0
    for t in range(NW1):
        h_t = allh_v[t, :]
        count = count + h_t
        before = before + h_t * _lt(iota * 0 + t, widv)

    padded = ((count + (BT - 1)) >> BT_LOG) << BT_LOG
    startx = _prefix(padded, iota) - padded
    ends = startx + padded
    carry = startx + before

    for c in range(PP1 // LANES):
        ev = ev_v[pl.ds(c * LANES, LANES)]
        rank = iota * 0
        chist = iota * 0
        for e in range(E):
            evec = iota * 0 + e
            mi = _eq(ev, evec)
            cs = _prefix(mi, iota)
            rank = rank + mi * (cs - 1)
            chist = chist + _vgather(cs, c15) * _eq(iota, evec)
        dst = _vgather(carry, ev) + rank
        carry = carry + chist
        dsts_v[pl.ds(c * LANES, LANES)] = dst
        row, half = c // 8, (c % 8) * LANES
        dst2d[row, pl.ds(half, LANES)] = dst

    pltpu.sync_copy(dsts_v, pos_hbm.at[pl.ds(base, PP1)])
    for j in range(2):
        pltpu.async_copy(wv2d.at[j], ws_hbm.at[dst2d.at[j]], sem).wait()

    @pl.when(wid == 0)
    def _():
        for cc in range(NB_PAD // LANES):
            bid = (cc * LANES + iota) * BT
            acc = iota * 0
            for e in range(E):
                end_splat = _vgather(ends, iota * 0 + e)
                acc = acc + (1 - _lt(bid, end_splat))
            beh_v[pl.ds(cc * LANES, LANES)] = jnp.minimum(acc, E - 1)
        pltpu.sync_copy(beh_v, be_hbm)


def _gather_body(hidden_hbm, pos_hbm, xs_hbm,
                 pos_v, tok2d, dst2d, rows_v, sem):
    wid = lax.axis_index("s") * 2 + lax.axis_index("c")
    base = wid * PP
    iota = lax.iota(jnp.int32, LANES)
    widv = iota * 0 + wid

    pltpu.sync_copy(pos_hbm.at[pl.ds(base, PP)], pos_v)
    for c in range(PP // LANES):
        row, half = c // 2, (c % 2) * LANES
        dst2d[row, pl.ds(half, LANES)] = pos_v[pl.ds(c * LANES, LANES)]
        tok = (widv * PP + c * LANES + iota) & (T - 1)
        tok2d[row, pl.ds(half, LANES)] = tok

    for j in range(4):
        pltpu.async_copy(hidden_hbm.at[tok2d.at[j]], rows_v, sem).wait()
        pltpu.async_copy(rows_v, xs_hbm.at[dst2d.at[j]], sem).wait()


def _ffn_body(be_ref, ws_ref, xs_ref, gu_ref, dp_ref, ys_ref):
    x = xs_ref[...].astype(jnp.bfloat16)
    gate = lax.dot_general(x, gu_ref[0, :D_FF, :],
                           (((1,), (1,)), ((), ())),
                           preferred_element_type=jnp.float32)
    up = lax.dot_general(x, gu_ref[0, D_FF:, :],
                         (((1,), (1,)), ((), ())),
                         preferred_element_type=jnp.float32)
    h = (jax.nn.silu(gate) * up).astype(jnp.bfloat16)
    eo = lax.dot_general(h, dp_ref[0],
                         (((1,), (1,)), ((), ())),
                         preferred_element_type=jnp.float32)
    w = ws_ref[0, 0, :]
    ys_ref[...] = w[:, None] * eo


def _combine_body(ys_hbm, pos_hbm, out_hbm, p0_v, p1_v, a0_v, a1_v, o_v,
                  sem, sem2):
    wid = lax.axis_index("s") * 2 + lax.axis_index("c")
    tb = wid * TPT
    pltpu.sync_copy(pos_hbm.at[pl.ds(tb, TPT)], p0_v)
    pltpu.sync_copy(pos_hbm.at[pl.ds(T + tb, TPT)], p1_v)
    for j in range(TPT // LANES):
        cp0 = pltpu.async_copy(
            ys_hbm.at[p0_v.at[pl.ds(j * LANES, LANES)]], a0_v, sem)
        cp1 = pltpu.async_copy(
            ys_hbm.at[p1_v.at[pl.ds(j * LANES, LANES)]], a1_v, sem2)
        cp0.wait()
        cp1.wait()
        for t in range(LANES):
            def _add(k, carry, t=t):
                col = k * (8 * LANES)
                for u in range(8):
                    sl = pl.ds(col + u * LANES, LANES)
                    o_v[t, sl] = a0_v[t, sl] + a1_v[t, sl]
                return carry
            lax.fori_loop(0, D_MODEL // (8 * LANES), _add, 0)
        pltpu.sync_copy(o_v, out_hbm.at[pl.ds(tb + j * LANES, LANES)])


def kernel(hidden_states, topk_indices, topk_weights, gate_up_proj, down_proj):
    idx_cm = topk_indices.astype(jnp.int32).T.reshape(NPAIR)
    wts_cm = topk_weights.T.reshape(NPAIR)
    gu_bf = gate_up_proj.astype(jnp.bfloat16)
    dp_bf = down_proj.astype(jnp.bfloat16)

    mesh1 = plsc.VectorSubcoreMesh(
        core_axis_name="c", subcore_axis_name="s", num_cores=1)
    mesh2 = plsc.VectorSubcoreMesh(
        core_axis_name="c", subcore_axis_name="s", num_cores=2)

    route = pl.kernel(
        _route_body,
        out_type=[
            jax.ShapeDtypeStruct((CAP,), jnp.float32),
            jax.ShapeDtypeStruct((NB_PAD,), jnp.int32),
            jax.ShapeDtypeStruct((NPAIR,), jnp.int32),
            jax.ShapeDtypeStruct((NW1, LANES), jnp.int32),
        ],
        mesh=mesh1,
        scratch_types=[
            pltpu.VMEM((PP1,), jnp.int32),
            pltpu.VMEM((PP1,), jnp.int32),
            pltpu.VMEM((LANES,), jnp.int32),
            pltpu.VMEM((NW1, LANES), jnp.int32),
            pltpu.VMEM((NB_PAD,), jnp.int32),
            pltpu.VMEM((2, PP1 // 2), jnp.int32),
            pltpu.VMEM((2, PP1 // 2), jnp.float32),
            pltpu.SemaphoreType.DMA,
        ],
    )
    ws, be, pos, _ = route(idx_cm, wts_cm)

    gather = pl.kernel(
        _gather_body,
        out_type=jax.ShapeDtypeStruct((CAP, D_MODEL), jnp.float32),
        mesh=mesh2,
        scratch_types=[
            pltpu.VMEM((PP,), jnp.int32),
            pltpu.VMEM((4, PP // 4), jnp.int32),
            pltpu.VMEM((4, PP // 4), jnp.int32),
            pltpu.VMEM((PP // 4, D_MODEL), jnp.float32),
            pltpu.SemaphoreType.DMA,
        ],
    )
    xs = gather(hidden_states, pos)

    grid_spec = pltpu.PrefetchScalarGridSpec(
        num_scalar_prefetch=1,
        grid=(NB,),
        in_specs=[
            pl.BlockSpec((1, 1, BT), lambda i, be_r: (i, 0, 0)),
            pl.BlockSpec((BT, D_MODEL), lambda i, be_r: (i, 0)),
            pl.BlockSpec((1, 2 * D_FF, D_MODEL), lambda i, be_r: (be_r[i], 0, 0)),
            pl.BlockSpec((1, D_MODEL, D_FF), lambda i, be_r: (be_r[i], 0, 0)),
        ],
        out_specs=pl.BlockSpec((BT, D_MODEL), lambda i, be_r: (i, 0)),
    )
    ys = pl.pallas_call(
        _ffn_body,
        grid_spec=grid_spec,
        out_shape=jax.ShapeDtypeStruct((CAP, D_MODEL), jnp.float32),
    )(be, ws.reshape(NB, 1, BT), xs, gu_bf, dp_bf)

    combine = pl.kernel(
        _combine_body,
        out_type=jax.ShapeDtypeStruct((T, D_MODEL), jnp.float32),
        mesh=mesh2,
        scratch_types=[
            pltpu.VMEM((TPT,), jnp.int32),
            pltpu.VMEM((TPT,), jnp.int32),
            pltpu.VMEM((LANES, D_MODEL), jnp.float32),
            pltpu.VMEM((LANES, D_MODEL), jnp.float32),
            pltpu.VMEM((LANES, D_MODEL), jnp.float32),
            pltpu.SemaphoreType.DMA,
            pltpu.SemaphoreType.DMA,
        ],
    )
    return combine(ys, pos)
